# Initial kernel scaffold; baseline (speedup 1.0000x reference)
import jax
import jax.numpy as jnp
from jax.experimental import pallas as pl


def kernel(x, edge_index, W1, b1, W2, b2, P1, pb1, P2, pb2, alpha1, alpha2):
    raise NotImplementedError("write your pallas kernel here")


# speedup vs baseline: 19.1500x; 19.1500x over previous
import functools

import jax
import jax.numpy as jnp
from jax import lax
from jax.experimental import pallas as pl
from jax.experimental.pallas import tpu as pltpu
from jax.experimental.pallas import tpu_sc as plsc

N = 10000
NP = 10240
E = 320000
CHUNK = 128
NC, NS = 2, 16
NW = NC * NS
K = -(-E // (NW * CHUNK))
EP = NW * K * CHUNK
RPT = NP // NS
BLK = 2048


def _sc_edge_accumulate(table, src_t, dst_t, zeros, f):
  mesh = plsc.VectorSubcoreMesh(core_axis_name="c", subcore_axis_name="s")

  @functools.partial(
      pl.kernel,
      out_type=jax.ShapeDtypeStruct((NC * NP, f), jnp.float32),
      mesh=mesh,
      scratch_types=[
          pltpu.VMEM((K, CHUNK), jnp.int32),
          pltpu.VMEM((K, CHUNK), jnp.int32),
          pltpu.VMEM((CHUNK, f), jnp.float32),
          pltpu.VMEM_SHARED((NP, f), jnp.float32),
      ],
      compiler_params=pltpu.CompilerParams(use_tc_tiling_on_sc=False),
  )
  def k(table_hbm, src_hbm, dst_hbm, zeros_hbm, out_hbm,
        src_v, dst_v, buf_v, acc_sh):
    c = lax.axis_index("c")
    s = lax.axis_index("s")
    wid = c * NS + s
    r0 = s * RPT

    @pl.when(c == 0)
    def _():
      pltpu.sync_copy(table_hbm.at[pl.ds(r0, RPT)], acc_sh.at[pl.ds(r0, RPT)])

    @pl.when(c != 0)
    def _():
      pltpu.sync_copy(zeros_hbm.at[pl.ds(r0, RPT)], acc_sh.at[pl.ds(r0, RPT)])

    pltpu.sync_copy(src_hbm.at[wid], src_v)
    pltpu.sync_copy(dst_hbm.at[wid], dst_v)
    plsc.subcore_barrier()

    @pl.loop(0, K)
    def _(j):
      pltpu.sync_copy(table_hbm.at[src_v.at[j]], buf_v)
      pltpu.sync_copy(buf_v, acc_sh.at[dst_v.at[j]], add=True)

    plsc.subcore_barrier()
    pltpu.sync_copy(acc_sh.at[pl.ds(r0, RPT)],
                    out_hbm.at[pl.ds(c * NP + r0, RPT)])

  return k(table, src_t, dst_t, zeros).reshape(NC, NP, f)


def _silu(v):
  return v * jax.nn.sigmoid(v)


def _row_grid(n_in, f_list, n_out, g_list):
  del n_in, n_out
  ins = [pl.BlockSpec((BLK, w), lambda i: (i, 0)) if w else None
         for w in f_list]
  outs = [pl.BlockSpec((BLK, w), lambda i: (i, 0)) for w in g_list]
  return ins, outs


def _tc_pre(xp, w1, p1, pb1, cnt0, cnt1):
  def body(x_ref, w1_ref, p1_ref, pb1_ref, c0_ref, c1_ref, s1_ref, xr1_ref):
    dinv = lax.rsqrt(c0_ref[:, 0:1] + c1_ref[:, 0:1])
    xb = x_ref[...]
    h1 = jnp.dot(xb, w1_ref[...], preferred_element_type=jnp.float32)
    s1_ref[...] = h1 * dinv
    xr1_ref[...] = _silu(
        jnp.dot(xb, p1_ref[...], preferred_element_type=jnp.float32)
        + pb1_ref[...])

  full = lambda shape: pl.BlockSpec(shape, lambda i: (0, 0))
  blk = lambda w: pl.BlockSpec((BLK, w), lambda i: (i, 0))
  return pl.pallas_call(
      body,
      grid=(NP // BLK,),
      in_specs=[blk(128), full((128, 64)), full((128, 64)), full((1, 64)),
                blk(16), blk(16)],
      out_specs=[blk(64), blk(64)],
      out_shape=[jax.ShapeDtypeStruct((NP, 64), jnp.float32),
                 jax.ShapeDtypeStruct((NP, 64), jnp.float32)],
  )(xp, w1, p1, pb1, cnt0, cnt1)


def _tc_mid(a0, a1, cnt0, cnt1, b1, xr1, al1, w2, p2, pb2):
  def body(a0_ref, a1_ref, c0_ref, c1_ref, b1_ref, xr1_ref, al1_ref,
           w2_ref, p2_ref, pb2_ref, s2_ref, xr2_ref):
    dinv = lax.rsqrt(c0_ref[:, 0:1] + c1_ref[:, 0:1])
    out1 = (a0_ref[...] + a1_ref[...]) * dinv + b1_ref[...]
    h = _silu(out1) + al1_ref[0, 0] * xr1_ref[...]
    s2_ref[...] = jnp.dot(
        h, w2_ref[...], preferred_element_type=jnp.float32) * dinv
    xr2_ref[...] = _silu(
        jnp.dot(h, p2_ref[...], preferred_element_type=jnp.float32)
        + pb2_ref[...])

  full = lambda shape: pl.BlockSpec(shape, lambda i: (0, 0))
  blk = lambda w: pl.BlockSpec((BLK, w), lambda i: (i, 0))
  return pl.pallas_call(
      body,
      grid=(NP // BLK,),
      in_specs=[blk(64), blk(64), blk(16), blk(16), full((1, 64)), blk(64),
                full((1, 1)), full((64, 16)), full((64, 16)), full((1, 16))],
      out_specs=[blk(16), blk(16)],
      out_shape=[jax.ShapeDtypeStruct((NP, 16), jnp.float32),
                 jax.ShapeDtypeStruct((NP, 16), jnp.float32)],
  )(a0, a1, cnt0, cnt1, b1, xr1, al1, w2, p2, pb2)


def _tc_post(a0, a1, cnt0, cnt1, b2, xr2, al2):
  def body(a0_ref, a1_ref, c0_ref, c1_ref, b2_ref, xr2_ref, al2_ref, z_ref):
    dinv = lax.rsqrt(c0_ref[:, 0:1] + c1_ref[:, 0:1])
    z_ref[...] = ((a0_ref[...] + a1_ref[...]) * dinv + b2_ref[...]
                  + al2_ref[0, 0] * xr2_ref[...])

  full = lambda shape: pl.BlockSpec(shape, lambda i: (0, 0))
  blk = lambda w: pl.BlockSpec((BLK, w), lambda i: (i, 0))
  return pl.pallas_call(
      body,
      grid=(NP // BLK,),
      in_specs=[blk(16), blk(16), blk(16), blk(16), full((1, 16)), blk(16),
                full((1, 1))],
      out_specs=blk(16),
      out_shape=jax.ShapeDtypeStruct((NP, 16), jnp.float32),
  )(a0, a1, cnt0, cnt1, b2, xr2, al2)


def kernel(x, edge_index, W1, b1, W2, b2, P1, pb1, P2, pb2, alpha1, alpha2):
  ei = edge_index.astype(jnp.int32)
  pad = jnp.full((EP - E,), N, jnp.int32)
  src_t = jnp.concatenate([ei[0], pad]).reshape(NW, K, CHUNK)
  dst_t = jnp.concatenate([ei[1], pad]).reshape(NW, K, CHUNK)

  xp = jnp.pad(x, ((0, NP - N), (0, 0)))
  zeros16 = jnp.zeros((NP, 16), jnp.float32)
  zeros64 = jnp.zeros((NP, 64), jnp.float32)
  ones16 = jnp.ones((NP, 16), jnp.float32)

  cnt = _sc_edge_accumulate(ones16, src_t, dst_t, zeros16, 16)

  s1, xr1 = _tc_pre(xp, W1, P1, pb1.reshape(1, 64), cnt[0], cnt[1])

  acc1 = _sc_edge_accumulate(s1, src_t, dst_t, zeros64, 64)

  s2, xr2 = _tc_mid(acc1[0], acc1[1], cnt[0], cnt[1], b1.reshape(1, 64),
                    xr1, alpha1.reshape(1, 1), W2, P2, pb2.reshape(1, 16))

  acc2 = _sc_edge_accumulate(s2, src_t, dst_t, zeros16, 16)

  z = _tc_post(acc2[0], acc2[1], cnt[0], cnt[1], b2.reshape(1, 16),
               xr2, alpha2.reshape(1, 1))
  return z[:N]


# speedup vs baseline: 20.1617x; 1.0528x over previous
import functools

import jax
import jax.numpy as jnp
from jax import lax
from jax.experimental import pallas as pl
from jax.experimental.pallas import tpu as pltpu
from jax.experimental.pallas import tpu_sc as plsc

N = 10000
NP = 10240
E = 320000
CHUNK = 128
NC, NS = 2, 16
NW = NC * NS
K = 80
EP = NW * K * CHUNK
RPT = NP // NS
BLK = 2048


def _sc_edge_accumulate(table, src_t, dst_t, zeros, f):
  mesh = plsc.VectorSubcoreMesh(core_axis_name="c", subcore_axis_name="s")

  @functools.partial(
      pl.kernel,
      out_type=jax.ShapeDtypeStruct((NC * NP, f), jnp.float32),
      mesh=mesh,
      scratch_types=[
          pltpu.VMEM((K, CHUNK), jnp.int32),
          pltpu.VMEM((K, CHUNK), jnp.int32),
          pltpu.VMEM((CHUNK, f), jnp.float32),
          pltpu.VMEM((CHUNK, f), jnp.float32),
          pltpu.VMEM_SHARED((NP, f), jnp.float32),
          pltpu.SemaphoreType.DMA,
          pltpu.SemaphoreType.DMA,
      ],
      compiler_params=pltpu.CompilerParams(use_tc_tiling_on_sc=False),
  )
  def k(table_hbm, src_hbm, dst_hbm, zeros_hbm, out_hbm,
        src_v, dst_v, buf0, buf1, acc_sh, sem0, sem1):
    c = lax.axis_index("c")
    s = lax.axis_index("s")
    wid = c * NS + s
    r0 = s * RPT

    @pl.when(c == 0)
    def _():
      pltpu.sync_copy(table_hbm.at[pl.ds(r0, RPT)], acc_sh.at[pl.ds(r0, RPT)])

    @pl.when(c != 0)
    def _():
      pltpu.sync_copy(zeros_hbm.at[pl.ds(r0, RPT)], acc_sh.at[pl.ds(r0, RPT)])

    pltpu.sync_copy(src_hbm.at[wid], src_v)
    pltpu.sync_copy(dst_hbm.at[wid], dst_v)
    plsc.subcore_barrier()

    pltpu.async_copy(table_hbm.at[src_v.at[0]], buf0, sem0)

    @pl.loop(0, K, step=2)
    def _(j):
      pltpu.async_copy(table_hbm.at[src_v.at[j + 1]], buf1, sem1)
      pltpu.make_async_copy(table_hbm.at[src_v.at[j]], buf0, sem0).wait()
      pltpu.sync_copy(buf0, acc_sh.at[dst_v.at[j]], add=True)

      @pl.when(j + 2 < K)
      def _():
        pltpu.async_copy(table_hbm.at[src_v.at[j + 2]], buf0, sem0)

      pltpu.make_async_copy(table_hbm.at[src_v.at[j + 1]], buf1, sem1).wait()
      pltpu.sync_copy(buf1, acc_sh.at[dst_v.at[j + 1]], add=True)

    plsc.subcore_barrier()
    pltpu.sync_copy(acc_sh.at[pl.ds(r0, RPT)],
                    out_hbm.at[pl.ds(c * NP + r0, RPT)])

  return k(table, src_t, dst_t, zeros).reshape(NC, NP, f)


def _sc_degree_count(src_t, dst_t, ones, zeros):
  mesh = plsc.VectorSubcoreMesh(core_axis_name="c", subcore_axis_name="s")

  @functools.partial(
      pl.kernel,
      out_type=jax.ShapeDtypeStruct((NC * NP, 16), jnp.float32),
      mesh=mesh,
      scratch_types=[
          pltpu.VMEM((K, CHUNK), jnp.int32),
          pltpu.VMEM((CHUNK, 16), jnp.float32),
          pltpu.VMEM_SHARED((NP, 16), jnp.float32),
          pltpu.SemaphoreType.DMA,
      ],
      compiler_params=pltpu.CompilerParams(use_tc_tiling_on_sc=False),
  )
  def k(ones_hbm, src_hbm, dst_hbm, zeros_hbm, out_hbm,
        dst_v, ones_v, acc_sh, sem):
    c = lax.axis_index("c")
    s = lax.axis_index("s")
    wid = c * NS + s
    r0 = s * RPT

    @pl.when(c == 0)
    def _():
      pltpu.sync_copy(ones_hbm.at[pl.ds(r0, RPT)], acc_sh.at[pl.ds(r0, RPT)])

    @pl.when(c != 0)
    def _():
      pltpu.sync_copy(zeros_hbm.at[pl.ds(r0, RPT)], acc_sh.at[pl.ds(r0, RPT)])

    pltpu.sync_copy(ones_hbm.at[pl.ds(0, CHUNK)], ones_v)
    pltpu.sync_copy(dst_hbm.at[wid], dst_v)
    plsc.subcore_barrier()

    @pl.loop(0, K, step=4)
    def _(j):
      for u in range(4):
        pltpu.async_copy(ones_v, acc_sh.at[dst_v.at[j + u]], sem, add=True)
      for u in range(4):
        pltpu.make_async_copy(ones_v, acc_sh.at[dst_v.at[j + u]], sem).wait()

    plsc.subcore_barrier()
    pltpu.sync_copy(acc_sh.at[pl.ds(r0, RPT)],
                    out_hbm.at[pl.ds(c * NP + r0, RPT)])

  return k(ones, src_t, dst_t, zeros).reshape(NC, NP, 16)


def _silu(v):
  return v * jax.nn.sigmoid(v)


def _tc_pre(xp, w1, p1, pb1, cnt0, cnt1):
  def body(x_ref, w1_ref, p1_ref, pb1_ref, c0_ref, c1_ref, s1_ref, xr1_ref):
    dinv = lax.rsqrt(c0_ref[:, 0:1] + c1_ref[:, 0:1])
    xb = x_ref[...]
    h1 = jnp.dot(xb, w1_ref[...], preferred_element_type=jnp.float32)
    s1_ref[...] = h1 * dinv
    xr1_ref[...] = _silu(
        jnp.dot(xb, p1_ref[...], preferred_element_type=jnp.float32)
        + pb1_ref[...])

  full = lambda shape: pl.BlockSpec(shape, lambda i: (0, 0))
  blk = lambda w: pl.BlockSpec((BLK, w), lambda i: (i, 0))
  return pl.pallas_call(
      body,
      grid=(NP // BLK,),
      in_specs=[blk(128), full((128, 64)), full((128, 64)), full((1, 64)),
                blk(16), blk(16)],
      out_specs=[blk(64), blk(64)],
      out_shape=[jax.ShapeDtypeStruct((NP, 64), jnp.float32),
                 jax.ShapeDtypeStruct((NP, 64), jnp.float32)],
  )(xp, w1, p1, pb1, cnt0, cnt1)


def _tc_mid(a0, a1, cnt0, cnt1, b1, xr1, al1, w2, p2, pb2):
  def body(a0_ref, a1_ref, c0_ref, c1_ref, b1_ref, xr1_ref, al1_ref,
           w2_ref, p2_ref, pb2_ref, s2_ref, xr2_ref):
    dinv = lax.rsqrt(c0_ref[:, 0:1] + c1_ref[:, 0:1])
    out1 = (a0_ref[...] + a1_ref[...]) * dinv + b1_ref[...]
    h = _silu(out1) + al1_ref[0, 0] * xr1_ref[...]
    s2_ref[...] = jnp.dot(
        h, w2_ref[...], preferred_element_type=jnp.float32) * dinv
    xr2_ref[...] = _silu(
        jnp.dot(h, p2_ref[...], preferred_element_type=jnp.float32)
        + pb2_ref[...])

  full = lambda shape: pl.BlockSpec(shape, lambda i: (0, 0))
  blk = lambda w: pl.BlockSpec((BLK, w), lambda i: (i, 0))
  return pl.pallas_call(
      body,
      grid=(NP // BLK,),
      in_specs=[blk(64), blk(64), blk(16), blk(16), full((1, 64)), blk(64),
                full((1, 1)), full((64, 16)), full((64, 16)), full((1, 16))],
      out_specs=[blk(16), blk(16)],
      out_shape=[jax.ShapeDtypeStruct((NP, 16), jnp.float32),
                 jax.ShapeDtypeStruct((NP, 16), jnp.float32)],
  )(a0, a1, cnt0, cnt1, b1, xr1, al1, w2, p2, pb2)


def _tc_post(a0, a1, cnt0, cnt1, b2, xr2, al2):
  def body(a0_ref, a1_ref, c0_ref, c1_ref, b2_ref, xr2_ref, al2_ref, z_ref):
    dinv = lax.rsqrt(c0_ref[:, 0:1] + c1_ref[:, 0:1])
    z_ref[...] = ((a0_ref[...] + a1_ref[...]) * dinv + b2_ref[...]
                  + al2_ref[0, 0] * xr2_ref[...])

  full = lambda shape: pl.BlockSpec(shape, lambda i: (0, 0))
  blk = lambda w: pl.BlockSpec((BLK, w), lambda i: (i, 0))
  return pl.pallas_call(
      body,
      grid=(NP // BLK,),
      in_specs=[blk(16), blk(16), blk(16), blk(16), full((1, 16)), blk(16),
                full((1, 1))],
      out_specs=blk(16),
      out_shape=jax.ShapeDtypeStruct((NP, 16), jnp.float32),
  )(a0, a1, cnt0, cnt1, b2, xr2, al2)


def kernel(x, edge_index, W1, b1, W2, b2, P1, pb1, P2, pb2, alpha1, alpha2):
  ei = edge_index.astype(jnp.int32)
  pad = jnp.full((EP - E,), N, jnp.int32)
  src_t = jnp.concatenate([ei[0], pad]).reshape(NW, K, CHUNK)
  dst_t = jnp.concatenate([ei[1], pad]).reshape(NW, K, CHUNK)

  xp = jnp.pad(x, ((0, NP - N), (0, 0)))
  zeros16 = jnp.zeros((NP, 16), jnp.float32)
  zeros64 = jnp.zeros((NP, 64), jnp.float32)
  ones16 = jnp.ones((NP, 16), jnp.float32)

  cnt = _sc_degree_count(src_t, dst_t, ones16, zeros16)

  s1, xr1 = _tc_pre(xp, W1, P1, pb1.reshape(1, 64), cnt[0], cnt[1])

  acc1 = _sc_edge_accumulate(s1, src_t, dst_t, zeros64, 64)

  s2, xr2 = _tc_mid(acc1[0], acc1[1], cnt[0], cnt[1], b1.reshape(1, 64),
                    xr1, alpha1.reshape(1, 1), W2, P2, pb2.reshape(1, 16))

  acc2 = _sc_edge_accumulate(s2, src_t, dst_t, zeros16, 16)

  z = _tc_post(acc2[0], acc2[1], cnt[0], cnt[1], b2.reshape(1, 16),
               xr2, alpha2.reshape(1, 1))
  return z[:N]


# speedup vs baseline: 22.3860x; 1.1103x over previous
import functools

import jax
import jax.numpy as jnp
from jax import lax
from jax.experimental import pallas as pl
from jax.experimental.pallas import tpu as pltpu
from jax.experimental.pallas import tpu_sc as plsc

N = 10000
NP = 10240
E = 320000
CHUNK = 128
NC, NS = 2, 16
NW = NC * NS
K = 80
EP = NW * K * CHUNK
RPT = NP // NS
BLK = 2048


def _sc_edge_accumulate(table, src_t, dst_t, zeros, f):
  mesh = plsc.VectorSubcoreMesh(core_axis_name="c", subcore_axis_name="s")

  nbuf = 4

  @functools.partial(
      pl.kernel,
      out_type=[jax.ShapeDtypeStruct((NP, f), jnp.float32),
                jax.ShapeDtypeStruct((NP, f), jnp.float32)],
      mesh=mesh,
      scratch_types=[
          pltpu.VMEM((K, CHUNK), jnp.int32),
          pltpu.VMEM((K, CHUNK), jnp.int32),
          [pltpu.VMEM((CHUNK, f), jnp.float32)] * nbuf,
          [pltpu.SemaphoreType.DMA] * nbuf,
          pltpu.VMEM_SHARED((NP, f), jnp.float32),
      ],
      compiler_params=pltpu.CompilerParams(use_tc_tiling_on_sc=False),
  )
  def k(table_hbm, src_hbm, dst_hbm, zeros_hbm, out0_hbm, out1_hbm,
        src_v, dst_v, bufs, sems, acc_sh):
    c = lax.axis_index("c")
    s = lax.axis_index("s")
    wid = c * NS + s
    r0 = s * RPT

    @pl.when(c == 0)
    def _():
      pltpu.sync_copy(table_hbm.at[pl.ds(r0, RPT)], acc_sh.at[pl.ds(r0, RPT)])

    @pl.when(c != 0)
    def _():
      pltpu.sync_copy(zeros_hbm.at[pl.ds(r0, RPT)], acc_sh.at[pl.ds(r0, RPT)])

    pltpu.sync_copy(src_hbm.at[wid], src_v)
    pltpu.sync_copy(dst_hbm.at[wid], dst_v)
    plsc.subcore_barrier()

    for u in range(nbuf - 1):
      pltpu.async_copy(table_hbm.at[src_v.at[u]], bufs[u], sems[u])

    @pl.loop(0, K, step=nbuf)
    def _(j):
      for u in range(nbuf):
        jj = j + u
        pltpu.make_async_copy(
            table_hbm.at[src_v.at[jj]], bufs[u], sems[u]).wait()
        pltpu.sync_copy(bufs[u], acc_sh.at[dst_v.at[jj]], add=True)

        @pl.when(jj + nbuf - 1 < K)
        def _():
          pltpu.async_copy(table_hbm.at[src_v.at[jj + nbuf - 1]],
                           bufs[(u + nbuf - 1) % nbuf],
                           sems[(u + nbuf - 1) % nbuf])

    plsc.subcore_barrier()

    @pl.when(c == 0)
    def _():
      pltpu.sync_copy(acc_sh.at[pl.ds(r0, RPT)], out0_hbm.at[pl.ds(r0, RPT)])

    @pl.when(c != 0)
    def _():
      pltpu.sync_copy(acc_sh.at[pl.ds(r0, RPT)], out1_hbm.at[pl.ds(r0, RPT)])

  return k(table, src_t, dst_t, zeros)


def _sc_degree_count(src_t, dst_t, ones, zeros):
  mesh = plsc.VectorSubcoreMesh(core_axis_name="c", subcore_axis_name="s")

  @functools.partial(
      pl.kernel,
      out_type=[jax.ShapeDtypeStruct((NP, 16), jnp.float32),
                jax.ShapeDtypeStruct((NP, 16), jnp.float32)],
      mesh=mesh,
      scratch_types=[
          pltpu.VMEM((K, CHUNK), jnp.int32),
          pltpu.VMEM((CHUNK, 16), jnp.float32),
          pltpu.VMEM_SHARED((NP, 16), jnp.float32),
          pltpu.SemaphoreType.DMA,
      ],
      compiler_params=pltpu.CompilerParams(use_tc_tiling_on_sc=False),
  )
  def k(ones_hbm, src_hbm, dst_hbm, zeros_hbm, out0_hbm, out1_hbm,
        dst_v, ones_v, acc_sh, sem):
    c = lax.axis_index("c")
    s = lax.axis_index("s")
    wid = c * NS + s
    r0 = s * RPT

    @pl.when(c == 0)
    def _():
      pltpu.sync_copy(ones_hbm.at[pl.ds(r0, RPT)], acc_sh.at[pl.ds(r0, RPT)])

    @pl.when(c != 0)
    def _():
      pltpu.sync_copy(zeros_hbm.at[pl.ds(r0, RPT)], acc_sh.at[pl.ds(r0, RPT)])

    pltpu.sync_copy(ones_hbm.at[pl.ds(0, CHUNK)], ones_v)
    pltpu.sync_copy(dst_hbm.at[wid], dst_v)
    plsc.subcore_barrier()

    @pl.loop(0, K, step=4)
    def _(j):
      for u in range(4):
        pltpu.async_copy(ones_v, acc_sh.at[dst_v.at[j + u]], sem, add=True)
      for u in range(4):
        pltpu.make_async_copy(ones_v, acc_sh.at[dst_v.at[j + u]], sem).wait()

    plsc.subcore_barrier()

    @pl.when(c == 0)
    def _():
      pltpu.sync_copy(acc_sh.at[pl.ds(r0, RPT)], out0_hbm.at[pl.ds(r0, RPT)])

    @pl.when(c != 0)
    def _():
      pltpu.sync_copy(acc_sh.at[pl.ds(r0, RPT)], out1_hbm.at[pl.ds(r0, RPT)])

  return k(ones, src_t, dst_t, zeros)


def _silu(v):
  return v * jax.nn.sigmoid(v)


def _tc_pre(xp, w1, p1, pb1, cnt0, cnt1):
  def body(x_ref, w1_ref, p1_ref, pb1_ref, c0_ref, c1_ref, s1_ref, xr1_ref):
    dinv = lax.rsqrt(c0_ref[:, 0:1] + c1_ref[:, 0:1])
    xb = x_ref[...]
    h1 = jnp.dot(xb, w1_ref[...], preferred_element_type=jnp.float32)
    s1_ref[...] = h1 * dinv
    xr1_ref[...] = _silu(
        jnp.dot(xb, p1_ref[...], preferred_element_type=jnp.float32)
        + pb1_ref[...])

  full = lambda shape: pl.BlockSpec(shape, lambda i: (0, 0))
  blk = lambda w: pl.BlockSpec((BLK, w), lambda i: (i, 0))
  return pl.pallas_call(
      body,
      grid=(NP // BLK,),
      in_specs=[blk(128), full((128, 64)), full((128, 64)), full((1, 64)),
                blk(16), blk(16)],
      out_specs=[blk(64), blk(64)],
      out_shape=[jax.ShapeDtypeStruct((NP, 64), jnp.float32),
                 jax.ShapeDtypeStruct((NP, 64), jnp.float32)],
  )(xp, w1, p1, pb1, cnt0, cnt1)


def _tc_mid(a0, a1, cnt0, cnt1, b1, xr1, al1, w2, p2, pb2):
  def body(a0_ref, a1_ref, c0_ref, c1_ref, b1_ref, xr1_ref, al1_ref,
           w2_ref, p2_ref, pb2_ref, s2_ref, xr2_ref):
    dinv = lax.rsqrt(c0_ref[:, 0:1] + c1_ref[:, 0:1])
    out1 = (a0_ref[...] + a1_ref[...]) * dinv + b1_ref[...]
    h = _silu(out1) + al1_ref[0, 0] * xr1_ref[...]
    s2_ref[...] = jnp.dot(
        h, w2_ref[...], preferred_element_type=jnp.float32) * dinv
    xr2_ref[...] = _silu(
        jnp.dot(h, p2_ref[...], preferred_element_type=jnp.float32)
        + pb2_ref[...])

  full = lambda shape: pl.BlockSpec(shape, lambda i: (0, 0))
  blk = lambda w: pl.BlockSpec((BLK, w), lambda i: (i, 0))
  return pl.pallas_call(
      body,
      grid=(NP // BLK,),
      in_specs=[blk(64), blk(64), blk(16), blk(16), full((1, 64)), blk(64),
                full((1, 1)), full((64, 16)), full((64, 16)), full((1, 16))],
      out_specs=[blk(16), blk(16)],
      out_shape=[jax.ShapeDtypeStruct((NP, 16), jnp.float32),
                 jax.ShapeDtypeStruct((NP, 16), jnp.float32)],
  )(a0, a1, cnt0, cnt1, b1, xr1, al1, w2, p2, pb2)


def _tc_post(a0, a1, cnt0, cnt1, b2, xr2, al2):
  def body(a0_ref, a1_ref, c0_ref, c1_ref, b2_ref, xr2_ref, al2_ref, z_ref):
    dinv = lax.rsqrt(c0_ref[:, 0:1] + c1_ref[:, 0:1])
    z_ref[...] = ((a0_ref[...] + a1_ref[...]) * dinv + b2_ref[...]
                  + al2_ref[0, 0] * xr2_ref[...])

  full = lambda shape: pl.BlockSpec(shape, lambda i: (0, 0))
  blk = lambda w: pl.BlockSpec((BLK, w), lambda i: (i, 0))
  return pl.pallas_call(
      body,
      grid=(NP // BLK,),
      in_specs=[blk(16), blk(16), blk(16), blk(16), full((1, 16)), blk(16),
                full((1, 1))],
      out_specs=blk(16),
      out_shape=jax.ShapeDtypeStruct((NP, 16), jnp.float32),
  )(a0, a1, cnt0, cnt1, b2, xr2, al2)


def kernel(x, edge_index, W1, b1, W2, b2, P1, pb1, P2, pb2, alpha1, alpha2):
  ei = edge_index.astype(jnp.int32)
  pad = jnp.full((EP - E,), N, jnp.int32)
  src_t = jnp.concatenate([ei[0], pad]).reshape(NW, K, CHUNK)
  dst_t = jnp.concatenate([ei[1], pad]).reshape(NW, K, CHUNK)

  xp = jnp.pad(x, ((0, NP - N), (0, 0)))
  zeros16 = jnp.zeros((NP, 16), jnp.float32)
  zeros64 = jnp.zeros((NP, 64), jnp.float32)
  ones16 = jnp.ones((NP, 16), jnp.float32)

  cnt0, cnt1 = _sc_degree_count(src_t, dst_t, ones16, zeros16)

  s1, xr1 = _tc_pre(xp, W1, P1, pb1.reshape(1, 64), cnt0, cnt1)

  a10, a11 = _sc_edge_accumulate(s1, src_t, dst_t, zeros64, 64)

  s2, xr2 = _tc_mid(a10, a11, cnt0, cnt1, b1.reshape(1, 64),
                    xr1, alpha1.reshape(1, 1), W2, P2, pb2.reshape(1, 16))

  a20, a21 = _sc_edge_accumulate(s2, src_t, dst_t, zeros16, 16)

  z = _tc_post(a20, a21, cnt0, cnt1, b2.reshape(1, 16),
               xr2, alpha2.reshape(1, 1))
  return z[:N]


# speedup vs baseline: 24.8663x; 1.1108x over previous
import functools

import jax
import jax.numpy as jnp
from jax import lax
from jax.experimental import pallas as pl
from jax.experimental.pallas import tpu as pltpu
from jax.experimental.pallas import tpu_sc as plsc

N = 10000
NP = 10240
E = 320000
CHUNK = 128
NC, NS = 2, 16
KT = 160
CT = NS * KT + 128
EP = CT * CHUNK
RPT = NP // NS
BLK = 2048
NBUF = 4

SPLIT_DEG = (92, 68)
SPLIT_F64 = (128, 32)
SPLIT_F16 = (108, 52)


def _chunk_starts(c, s, ka, kb):
  return jnp.where(c == 0, s * ka, NS * ka + s * kb)


def _sc_edge_accumulate(table, src_t, dst_t, zeros, f, split):
  mesh = plsc.VectorSubcoreMesh(core_axis_name="c", subcore_axis_name="s")
  ka, kb = split
  kmax = max(ka, kb)

  @functools.partial(
      pl.kernel,
      out_type=[jax.ShapeDtypeStruct((NP, f), jnp.float32),
                jax.ShapeDtypeStruct((NP, f), jnp.float32)],
      mesh=mesh,
      scratch_types=[
          pltpu.VMEM((kmax, CHUNK), jnp.int32),
          pltpu.VMEM((kmax, CHUNK), jnp.int32),
          [pltpu.VMEM((CHUNK, f), jnp.float32)] * NBUF,
          [pltpu.SemaphoreType.DMA] * NBUF,
          pltpu.VMEM_SHARED((NP, f), jnp.float32),
      ],
      compiler_params=pltpu.CompilerParams(use_tc_tiling_on_sc=False),
  )
  def k(table_hbm, src_hbm, dst_hbm, zeros_hbm, out0_hbm, out1_hbm,
        src_v, dst_v, bufs, sems, acc_sh):
    c = lax.axis_index("c")
    s = lax.axis_index("s")
    r0 = s * RPT
    c0 = _chunk_starts(c, s, ka, kb)
    kc = jnp.where(c == 0, ka, kb)

    @pl.when(c == 0)
    def _():
      pltpu.sync_copy(table_hbm.at[pl.ds(r0, RPT)], acc_sh.at[pl.ds(r0, RPT)])

    @pl.when(c != 0)
    def _():
      pltpu.sync_copy(zeros_hbm.at[pl.ds(r0, RPT)], acc_sh.at[pl.ds(r0, RPT)])

    pltpu.sync_copy(src_hbm.at[pl.ds(c0, kmax)], src_v)
    pltpu.sync_copy(dst_hbm.at[pl.ds(c0, kmax)], dst_v)
    plsc.subcore_barrier()

    for u in range(NBUF - 1):
      pltpu.async_copy(table_hbm.at[src_v.at[u]], bufs[u], sems[u])

    @pl.loop(0, kc, step=NBUF)
    def _(j):
      for u in range(NBUF):
        jj = j + u
        pltpu.make_async_copy(
            table_hbm.at[src_v.at[jj]], bufs[u], sems[u]).wait()
        pltpu.sync_copy(bufs[u], acc_sh.at[dst_v.at[jj]], add=True)

        @pl.when(jj + NBUF - 1 < kc)
        def _():
          pltpu.async_copy(table_hbm.at[src_v.at[jj + NBUF - 1]],
                           bufs[(u + NBUF - 1) % NBUF],
                           sems[(u + NBUF - 1) % NBUF])

    plsc.subcore_barrier()

    @pl.when(c == 0)
    def _():
      pltpu.sync_copy(acc_sh.at[pl.ds(r0, RPT)], out0_hbm.at[pl.ds(r0, RPT)])

    @pl.when(c != 0)
    def _():
      pltpu.sync_copy(acc_sh.at[pl.ds(r0, RPT)], out1_hbm.at[pl.ds(r0, RPT)])

  return k(table, src_t, dst_t, zeros)


def _sc_degree_count(src_t, dst_t, ones, zeros):
  mesh = plsc.VectorSubcoreMesh(core_axis_name="c", subcore_axis_name="s")
  ka, kb = SPLIT_DEG
  kmax = max(ka, kb)

  @functools.partial(
      pl.kernel,
      out_type=[jax.ShapeDtypeStruct((NP, 16), jnp.float32),
                jax.ShapeDtypeStruct((NP, 16), jnp.float32)],
      mesh=mesh,
      scratch_types=[
          pltpu.VMEM((kmax, CHUNK), jnp.int32),
          pltpu.VMEM((CHUNK, 16), jnp.float32),
          pltpu.VMEM_SHARED((NP, 16), jnp.float32),
          pltpu.SemaphoreType.DMA,
      ],
      compiler_params=pltpu.CompilerParams(use_tc_tiling_on_sc=False),
  )
  def k(ones_hbm, src_hbm, dst_hbm, zeros_hbm, out0_hbm, out1_hbm,
        dst_v, ones_v, acc_sh, sem):
    c = lax.axis_index("c")
    s = lax.axis_index("s")
    r0 = s * RPT
    c0 = _chunk_starts(c, s, ka, kb)
    kc = jnp.where(c == 0, ka, kb)

    @pl.when(c == 0)
    def _():
      pltpu.sync_copy(ones_hbm.at[pl.ds(r0, RPT)], acc_sh.at[pl.ds(r0, RPT)])

    @pl.when(c != 0)
    def _():
      pltpu.sync_copy(zeros_hbm.at[pl.ds(r0, RPT)], acc_sh.at[pl.ds(r0, RPT)])

    pltpu.sync_copy(ones_hbm.at[pl.ds(0, CHUNK)], ones_v)
    pltpu.sync_copy(dst_hbm.at[pl.ds(c0, kmax)], dst_v)
    plsc.subcore_barrier()

    @pl.loop(0, kc, step=4)
    def _(j):
      for u in range(4):
        pltpu.async_copy(ones_v, acc_sh.at[dst_v.at[j + u]], sem, add=True)
      for u in range(4):
        pltpu.make_async_copy(ones_v, acc_sh.at[dst_v.at[j + u]], sem).wait()

    plsc.subcore_barrier()

    @pl.when(c == 0)
    def _():
      pltpu.sync_copy(acc_sh.at[pl.ds(r0, RPT)], out0_hbm.at[pl.ds(r0, RPT)])

    @pl.when(c != 0)
    def _():
      pltpu.sync_copy(acc_sh.at[pl.ds(r0, RPT)], out1_hbm.at[pl.ds(r0, RPT)])

  return k(ones, src_t, dst_t, zeros)


def _silu(v):
  return v * jax.nn.sigmoid(v)


def _tc_pre(xp, w1, p1, pb1, cnt0, cnt1):
  def body(x_ref, w1_ref, p1_ref, pb1_ref, c0_ref, c1_ref, s1_ref, xr1_ref):
    dinv = lax.rsqrt(c0_ref[:, 0:1] + c1_ref[:, 0:1])
    xb = x_ref[...]
    h1 = jnp.dot(xb, w1_ref[...], preferred_element_type=jnp.float32)
    s1_ref[...] = h1 * dinv
    xr1_ref[...] = _silu(
        jnp.dot(xb, p1_ref[...], preferred_element_type=jnp.float32)
        + pb1_ref[...])

  full = lambda shape: pl.BlockSpec(shape, lambda i: (0, 0))
  blk = lambda w: pl.BlockSpec((BLK, w), lambda i: (i, 0))
  return pl.pallas_call(
      body,
      grid=(NP // BLK,),
      in_specs=[blk(128), full((128, 64)), full((128, 64)), full((1, 64)),
                blk(16), blk(16)],
      out_specs=[blk(64), blk(64)],
      out_shape=[jax.ShapeDtypeStruct((NP, 64), jnp.float32),
                 jax.ShapeDtypeStruct((NP, 64), jnp.float32)],
  )(xp, w1, p1, pb1, cnt0, cnt1)


def _tc_mid(a0, a1, cnt0, cnt1, b1, xr1, al1, w2, p2, pb2):
  def body(a0_ref, a1_ref, c0_ref, c1_ref, b1_ref, xr1_ref, al1_ref,
           w2_ref, p2_ref, pb2_ref, s2_ref, xr2_ref):
    dinv = lax.rsqrt(c0_ref[:, 0:1] + c1_ref[:, 0:1])
    out1 = (a0_ref[...] + a1_ref[...]) * dinv + b1_ref[...]
    h = _silu(out1) + al1_ref[0, 0] * xr1_ref[...]
    s2_ref[...] = jnp.dot(
        h, w2_ref[...], preferred_element_type=jnp.float32) * dinv
    xr2_ref[...] = _silu(
        jnp.dot(h, p2_ref[...], preferred_element_type=jnp.float32)
        + pb2_ref[...])

  full = lambda shape: pl.BlockSpec(shape, lambda i: (0, 0))
  blk = lambda w: pl.BlockSpec((BLK, w), lambda i: (i, 0))
  return pl.pallas_call(
      body,
      grid=(NP // BLK,),
      in_specs=[blk(64), blk(64), blk(16), blk(16), full((1, 64)), blk(64),
                full((1, 1)), full((64, 16)), full((64, 16)), full((1, 16))],
      out_specs=[blk(16), blk(16)],
      out_shape=[jax.ShapeDtypeStruct((NP, 16), jnp.float32),
                 jax.ShapeDtypeStruct((NP, 16), jnp.float32)],
  )(a0, a1, cnt0, cnt1, b1, xr1, al1, w2, p2, pb2)


def _tc_post(a0, a1, cnt0, cnt1, b2, xr2, al2):
  def body(a0_ref, a1_ref, c0_ref, c1_ref, b2_ref, xr2_ref, al2_ref, z_ref):
    dinv = lax.rsqrt(c0_ref[:, 0:1] + c1_ref[:, 0:1])
    z_ref[...] = ((a0_ref[...] + a1_ref[...]) * dinv + b2_ref[...]
                  + al2_ref[0, 0] * xr2_ref[...])

  full = lambda shape: pl.BlockSpec(shape, lambda i: (0, 0))
  blk = lambda w: pl.BlockSpec((BLK, w), lambda i: (i, 0))
  return pl.pallas_call(
      body,
      grid=(NP // BLK,),
      in_specs=[blk(16), blk(16), blk(16), blk(16), full((1, 16)), blk(16),
                full((1, 1))],
      out_specs=blk(16),
      out_shape=jax.ShapeDtypeStruct((NP, 16), jnp.float32),
  )(a0, a1, cnt0, cnt1, b2, xr2, al2)


def kernel(x, edge_index, W1, b1, W2, b2, P1, pb1, P2, pb2, alpha1, alpha2):
  ei = edge_index.astype(jnp.int32)
  pad = jnp.full((EP - E,), N, jnp.int32)
  src_t = jnp.concatenate([ei[0], pad]).reshape(CT, CHUNK)
  dst_t = jnp.concatenate([ei[1], pad]).reshape(CT, CHUNK)

  xp = jnp.pad(x, ((0, NP - N), (0, 0)))
  zeros16 = jnp.zeros((NP, 16), jnp.float32)
  zeros64 = jnp.zeros((NP, 64), jnp.float32)
  ones16 = jnp.ones((NP, 16), jnp.float32)

  cnt0, cnt1 = _sc_degree_count(src_t, dst_t, ones16, zeros16)

  s1, xr1 = _tc_pre(xp, W1, P1, pb1.reshape(1, 64), cnt0, cnt1)

  a10, a11 = _sc_edge_accumulate(s1, src_t, dst_t, zeros64, 64, SPLIT_F64)

  s2, xr2 = _tc_mid(a10, a11, cnt0, cnt1, b1.reshape(1, 64),
                    xr1, alpha1.reshape(1, 1), W2, P2, pb2.reshape(1, 16))

  a20, a21 = _sc_edge_accumulate(s2, src_t, dst_t, zeros16, 16, SPLIT_F16)

  z = _tc_post(a20, a21, cnt0, cnt1, b2.reshape(1, 16),
               xr2, alpha2.reshape(1, 1))
  return z[:N]


# speedup vs baseline: 26.2536x; 1.0558x over previous
import functools

import jax
import jax.numpy as jnp
from jax import lax
from jax.experimental import pallas as pl
from jax.experimental.pallas import tpu as pltpu
from jax.experimental.pallas import tpu_sc as plsc

N = 10000
NP = 10240
E = 320000
CHUNK = 128
NC, NS = 2, 16
KT = 160
CT = NS * KT + 128
EP = CT * CHUNK
RPT = NP // NS
BLK = 2048
NBUF = 4

SPLIT_DEG = (100, 60)
SPLIT_F64 = (144, 16)
SPLIT_F16 = (112, 48)


def _chunk_starts(c, s, ka, kb):
  return jnp.where(c == 0, s * ka, NS * ka + s * kb)


def _sc_edge_accumulate(table, src_t, dst_t, zeros, f, split):
  mesh = plsc.VectorSubcoreMesh(core_axis_name="c", subcore_axis_name="s")
  ka, kb = split
  kmax = max(ka, kb)

  @functools.partial(
      pl.kernel,
      out_type=[jax.ShapeDtypeStruct((NP, f), jnp.float32),
                jax.ShapeDtypeStruct((NP, f), jnp.float32)],
      mesh=mesh,
      scratch_types=[
          pltpu.VMEM((kmax, CHUNK), jnp.int32),
          pltpu.VMEM((kmax, CHUNK), jnp.int32),
          [pltpu.VMEM((CHUNK, f), jnp.float32)] * NBUF,
          [pltpu.SemaphoreType.DMA] * NBUF,
          pltpu.VMEM_SHARED((NP, f), jnp.float32),
      ],
      compiler_params=pltpu.CompilerParams(use_tc_tiling_on_sc=False),
  )
  def k(table_hbm, src_hbm, dst_hbm, zeros_hbm, out0_hbm, out1_hbm,
        src_v, dst_v, bufs, sems, acc_sh):
    c = lax.axis_index("c")
    s = lax.axis_index("s")
    r0 = s * RPT
    c0 = _chunk_starts(c, s, ka, kb)
    kc = jnp.where(c == 0, ka, kb)

    @pl.when(c == 0)
    def _():
      pltpu.sync_copy(table_hbm.at[pl.ds(r0, RPT)], acc_sh.at[pl.ds(r0, RPT)])

    @pl.when(c != 0)
    def _():
      pltpu.sync_copy(zeros_hbm.at[pl.ds(r0, RPT)], acc_sh.at[pl.ds(r0, RPT)])

    pltpu.sync_copy(src_hbm.at[pl.ds(c0, kmax)], src_v)
    pltpu.sync_copy(dst_hbm.at[pl.ds(c0, kmax)], dst_v)
    plsc.subcore_barrier()

    for u in range(NBUF - 1):
      pltpu.async_copy(table_hbm.at[src_v.at[u]], bufs[u], sems[u])

    @pl.loop(0, kc, step=NBUF)
    def _(j):
      for u in range(NBUF):
        jj = j + u
        pltpu.make_async_copy(
            table_hbm.at[src_v.at[jj]], bufs[u], sems[u]).wait()
        pltpu.sync_copy(bufs[u], acc_sh.at[dst_v.at[jj]], add=True)

        @pl.when(jj + NBUF - 1 < kc)
        def _():
          pltpu.async_copy(table_hbm.at[src_v.at[jj + NBUF - 1]],
                           bufs[(u + NBUF - 1) % NBUF],
                           sems[(u + NBUF - 1) % NBUF])

    plsc.subcore_barrier()

    @pl.when(c == 0)
    def _():
      pltpu.sync_copy(acc_sh.at[pl.ds(r0, RPT)], out0_hbm.at[pl.ds(r0, RPT)])

    @pl.when(c != 0)
    def _():
      pltpu.sync_copy(acc_sh.at[pl.ds(r0, RPT)], out1_hbm.at[pl.ds(r0, RPT)])

  return k(table, src_t, dst_t, zeros)


def _sc_degree_count(src_t, dst_t, ones, zeros):
  mesh = plsc.VectorSubcoreMesh(core_axis_name="c", subcore_axis_name="s")
  ka, kb = SPLIT_DEG
  kmax = max(ka, kb)

  @functools.partial(
      pl.kernel,
      out_type=[jax.ShapeDtypeStruct((NP, 16), jnp.float32),
                jax.ShapeDtypeStruct((NP, 16), jnp.float32)],
      mesh=mesh,
      scratch_types=[
          pltpu.VMEM((kmax, CHUNK), jnp.int32),
          pltpu.VMEM((CHUNK, 16), jnp.float32),
          pltpu.VMEM_SHARED((NP, 16), jnp.float32),
          pltpu.SemaphoreType.DMA,
      ],
      compiler_params=pltpu.CompilerParams(use_tc_tiling_on_sc=False),
  )
  def k(ones_hbm, src_hbm, dst_hbm, zeros_hbm, out0_hbm, out1_hbm,
        dst_v, ones_v, acc_sh, sem):
    c = lax.axis_index("c")
    s = lax.axis_index("s")
    r0 = s * RPT
    c0 = _chunk_starts(c, s, ka, kb)
    kc = jnp.where(c == 0, ka, kb)

    @pl.when(c == 0)
    def _():
      pltpu.sync_copy(ones_hbm.at[pl.ds(r0, RPT)], acc_sh.at[pl.ds(r0, RPT)])

    @pl.when(c != 0)
    def _():
      pltpu.sync_copy(zeros_hbm.at[pl.ds(r0, RPT)], acc_sh.at[pl.ds(r0, RPT)])

    pltpu.sync_copy(ones_hbm.at[pl.ds(0, CHUNK)], ones_v)
    pltpu.sync_copy(dst_hbm.at[pl.ds(c0, kmax)], dst_v)
    plsc.subcore_barrier()

    @pl.loop(0, kc, step=4)
    def _(j):
      for u in range(4):
        pltpu.async_copy(ones_v, acc_sh.at[dst_v.at[j + u]], sem, add=True)
      for u in range(4):
        pltpu.make_async_copy(ones_v, acc_sh.at[dst_v.at[j + u]], sem).wait()

    plsc.subcore_barrier()

    @pl.when(c == 0)
    def _():
      pltpu.sync_copy(acc_sh.at[pl.ds(r0, RPT)], out0_hbm.at[pl.ds(r0, RPT)])

    @pl.when(c != 0)
    def _():
      pltpu.sync_copy(acc_sh.at[pl.ds(r0, RPT)], out1_hbm.at[pl.ds(r0, RPT)])

  return k(ones, src_t, dst_t, zeros)


def _silu(v):
  return v * jax.nn.sigmoid(v)


def _tc_mm1(xp, w1, p1, pb1):
  def body(x_ref, w1_ref, p1_ref, pb1_ref, h1_ref, xr1_ref):
    xb = x_ref[...]
    h1_ref[...] = jnp.dot(xb, w1_ref[...], preferred_element_type=jnp.float32)
    xr1_ref[...] = _silu(
        jnp.dot(xb, p1_ref[...], preferred_element_type=jnp.float32)
        + pb1_ref[...])

  full = lambda shape: pl.BlockSpec(shape, lambda i: (0, 0))
  blk = lambda w: pl.BlockSpec((BLK, w), lambda i: (i, 0))
  return pl.pallas_call(
      body,
      grid=(NP // BLK,),
      in_specs=[blk(128), full((128, 64)), full((128, 64)), full((1, 64))],
      out_specs=[blk(64), blk(64)],
      out_shape=[jax.ShapeDtypeStruct((NP, 64), jnp.float32),
                 jax.ShapeDtypeStruct((NP, 64), jnp.float32)],
  )(xp, w1, p1, pb1)


def _tc_scale1(h1, cnt0, cnt1):
  def body(h1_ref, c0_ref, c1_ref, s1_ref):
    dinv = lax.rsqrt(c0_ref[:, 0:1] + c1_ref[:, 0:1])
    s1_ref[...] = h1_ref[...] * dinv

  blk = lambda w: pl.BlockSpec((BLK, w), lambda i: (i, 0))
  return pl.pallas_call(
      body,
      grid=(NP // BLK,),
      in_specs=[blk(64), blk(16), blk(16)],
      out_specs=blk(64),
      out_shape=jax.ShapeDtypeStruct((NP, 64), jnp.float32),
  )(h1, cnt0, cnt1)


def _tc_mid(a0, a1, cnt0, cnt1, b1, xr1, al1, w2, p2, pb2):
  def body(a0_ref, a1_ref, c0_ref, c1_ref, b1_ref, xr1_ref, al1_ref,
           w2_ref, p2_ref, pb2_ref, s2_ref, xr2_ref):
    dinv = lax.rsqrt(c0_ref[:, 0:1] + c1_ref[:, 0:1])
    out1 = (a0_ref[...] + a1_ref[...]) * dinv + b1_ref[...]
    h = _silu(out1) + al1_ref[0, 0] * xr1_ref[...]
    s2_ref[...] = jnp.dot(
        h, w2_ref[...], preferred_element_type=jnp.float32) * dinv
    xr2_ref[...] = _silu(
        jnp.dot(h, p2_ref[...], preferred_element_type=jnp.float32)
        + pb2_ref[...])

  full = lambda shape: pl.BlockSpec(shape, lambda i: (0, 0))
  blk = lambda w: pl.BlockSpec((BLK, w), lambda i: (i, 0))
  return pl.pallas_call(
      body,
      grid=(NP // BLK,),
      in_specs=[blk(64), blk(64), blk(16), blk(16), full((1, 64)), blk(64),
                full((1, 1)), full((64, 16)), full((64, 16)), full((1, 16))],
      out_specs=[blk(16), blk(16)],
      out_shape=[jax.ShapeDtypeStruct((NP, 16), jnp.float32),
                 jax.ShapeDtypeStruct((NP, 16), jnp.float32)],
  )(a0, a1, cnt0, cnt1, b1, xr1, al1, w2, p2, pb2)


def _tc_post(a0, a1, cnt0, cnt1, b2, xr2, al2):
  def body(a0_ref, a1_ref, c0_ref, c1_ref, b2_ref, xr2_ref, al2_ref, z_ref):
    dinv = lax.rsqrt(c0_ref[:, 0:1] + c1_ref[:, 0:1])
    z_ref[...] = ((a0_ref[...] + a1_ref[...]) * dinv + b2_ref[...]
                  + al2_ref[0, 0] * xr2_ref[...])

  full = lambda shape: pl.BlockSpec(shape, lambda i: (0, 0))
  blk = lambda w: pl.BlockSpec((BLK, w), lambda i: (i, 0))
  return pl.pallas_call(
      body,
      grid=(NP // BLK,),
      in_specs=[blk(16), blk(16), blk(16), blk(16), full((1, 16)), blk(16),
                full((1, 1))],
      out_specs=blk(16),
      out_shape=jax.ShapeDtypeStruct((NP, 16), jnp.float32),
  )(a0, a1, cnt0, cnt1, b2, xr2, al2)


def kernel(x, edge_index, W1, b1, W2, b2, P1, pb1, P2, pb2, alpha1, alpha2):
  ei = edge_index.astype(jnp.int32)
  pad = jnp.full((EP - E,), N, jnp.int32)
  src_t = jnp.concatenate([ei[0], pad]).reshape(CT, CHUNK)
  dst_t = jnp.concatenate([ei[1], pad]).reshape(CT, CHUNK)

  xp = jnp.pad(x, ((0, NP - N), (0, 0)))
  zeros16 = jnp.zeros((NP, 16), jnp.float32)
  zeros64 = jnp.zeros((NP, 64), jnp.float32)
  ones16 = jnp.ones((NP, 16), jnp.float32)

  cnt0, cnt1 = _sc_degree_count(src_t, dst_t, ones16, zeros16)

  h1, xr1 = _tc_mm1(xp, W1, P1, pb1.reshape(1, 64))
  s1 = _tc_scale1(h1, cnt0, cnt1)

  a10, a11 = _sc_edge_accumulate(s1, src_t, dst_t, zeros64, 64, SPLIT_F64)

  s2, xr2 = _tc_mid(a10, a11, cnt0, cnt1, b1.reshape(1, 64),
                    xr1, alpha1.reshape(1, 1), W2, P2, pb2.reshape(1, 16))

  a20, a21 = _sc_edge_accumulate(s2, src_t, dst_t, zeros16, 16, SPLIT_F16)

  z = _tc_post(a20, a21, cnt0, cnt1, b2.reshape(1, 16),
               xr2, alpha2.reshape(1, 1))
  return z[:N]
